# baseline (speedup 1.0000x reference)
import functools

import jax
import jax.numpy as jnp
from jax import lax
from jax.experimental import pallas as pl
from jax.experimental.pallas import tpu as pltpu
from jax.experimental.pallas import tpu_sc as plsc

_CK = 16
_NS = 3


def _sc_body(t_hbm, i_hbm, a_hbm, e_hbm, out_hbm, ids_hbm,
             b0, b1, b2, bias_v, ids_v,
             is0, is1, is2, os0, os1, os2, idsem,
             *, B, H, seg_lens, tot, nw, nc):
    cid = lax.axis_index("c")
    sid = lax.axis_index("s")
    wid = sid * nc + cid

    bufs = (b0, b1, b2)
    isems = (is0, is1, is2)
    osems = (os0, os1, os2)

    pltpu.sync_copy(e_hbm.at[pl.ds(0, len(seg_lens)), :], bias_v)

    hbms = (t_hbm, i_hbm, a_hbm)
    ids_cps = []
    off = 0
    for m, lm in enumerate(seg_lens):
        x_hbm = hbms[m]
        rm = B * lm // nw
        in_base = wid * rm
        b = in_base // lm
        l0 = in_base - b * lm
        out_base = b * tot + off + l0
        nck = rm // _CK

        ids_off = sum(B * l // nw for l in seg_lens[:m])
        for i in range(rm // 16):
            ids_v[pl.ds(ids_off + i * 16, 16)] = jnp.full((16,), m, jnp.int32)
        cp = pltpu.make_async_copy(ids_v.at[pl.ds(ids_off, rm)],
                                   ids_hbm.at[pl.ds(out_base, rm)], idsem)
        cp.start()
        ids_cps.append(cp)

        def in_cp(kk, s, in_base=in_base, x_hbm=x_hbm):
            return pltpu.make_async_copy(
                x_hbm.at[pl.ds(in_base + kk * _CK, _CK), :], bufs[s], isems[s])

        def out_cp(kk, s, out_base=out_base):
            return pltpu.make_async_copy(
                bufs[s], out_hbm.at[pl.ds(out_base + kk * _CK, _CK), :], osems[s])

        in_cp(0, 0).start()

        def make_branch(s, nck=nck, m=m, in_cp=in_cp, out_cp=out_cp):
            s1 = (s + 1) % _NS

            def br(kk):
                in_cp(kk, s).wait()

                @pl.when(kk >= 2)
                def _():
                    out_cp(kk - 2, s1).wait()

                @pl.when(kk + 1 < nck)
                def _():
                    in_cp(kk + 1, s1).start()

                @plsc.parallel_loop(0, H // 16, unroll=8)
                def _(c):
                    sl = pl.ds(c * 16, 16)
                    bv = bias_v[m, sl]
                    for r in range(_CK):
                        bufs[s][r, sl] = bufs[s][r, sl] + bv

                out_cp(kk, s).start()
                return 0

            return br

        branches = tuple(make_branch(s) for s in range(_NS))

        def step(kk, _, branches=branches):
            lax.switch(kk % _NS, branches, kk)
            return 0

        lax.fori_loop(0, nck, step, 0)
        for j in (nck - 2, nck - 1):
            out_cp(j, j % _NS).wait()
        off += lm
    for cp in ids_cps:
        cp.wait()


def kernel(text, image, audio, modality_embed):
    B, l_t, H = text.shape
    l_i = image.shape[1]
    l_a = audio.shape[1]
    tot = l_t + l_i + l_a

    info = plsc.get_sparse_core_info()
    nc, ns = info.num_cores, info.num_subcores
    nw = nc * ns
    mesh = plsc.VectorSubcoreMesh(core_axis_name="c", subcore_axis_name="s")

    body = functools.partial(_sc_body, B=B, H=H, seg_lens=(l_t, l_i, l_a),
                             tot=tot, nw=nw, nc=nc)

    sck = pl.kernel(
        body,
        mesh=mesh,
        out_type=[
            jax.ShapeDtypeStruct((B * tot, H), jnp.float32),
            jax.ShapeDtypeStruct((B * tot,), jnp.int32),
        ],
        scratch_types=[
            pltpu.VMEM((_CK, H), jnp.float32),
            pltpu.VMEM((_CK, H), jnp.float32),
            pltpu.VMEM((_CK, H), jnp.float32),
            pltpu.VMEM((3, H), jnp.float32),
            pltpu.VMEM((B * tot // nw,), jnp.int32),
            pltpu.SemaphoreType.DMA,
            pltpu.SemaphoreType.DMA,
            pltpu.SemaphoreType.DMA,
            pltpu.SemaphoreType.DMA,
            pltpu.SemaphoreType.DMA,
            pltpu.SemaphoreType.DMA,
            pltpu.SemaphoreType.DMA,
        ],
    )
    out2, ids1 = sck(
        text.reshape(B * l_t, H),
        image.reshape(B * l_i, H),
        audio.reshape(B * l_a, H),
        modality_embed,
    )
    return out2.reshape(B, tot, H), ids1.reshape(B, tot)
